# Initial kernel scaffold; baseline (speedup 1.0000x reference)
import jax
import jax.numpy as jnp
from jax.experimental import pallas as pl


def kernel(x, s_log, alpha_log, delta_log, r_log):
    raise NotImplementedError("write your pallas kernel here")


# speedup vs baseline: 27.3513x; 27.3513x over previous
import jax
import jax.numpy as jnp
from jax.experimental import pallas as pl
from jax.experimental.pallas import tpu as pltpu

EPS = 1e-05
W = 128
B_BLK = 8


def _pcen_kernel(s_ref, alpha_ref, delta_ref, r_ref, x_ref, o_ref,
                 u_ref, p_ref, m_ref):
    t = pl.program_id(1)
    n_b, n_f, _ = x_ref.shape
    bf = n_b * n_f
    kk = s_ref.shape[1]

    x2 = x_ref[...].reshape(bf, W)

    @pl.when(t == 0)
    def _init():
        s_row = jnp.exp(s_ref[...])
        la_row = jnp.log1p(-s_row)
        i_iota = jax.lax.broadcasted_iota(jnp.int32, (W, W), 1)
        j_iota = jax.lax.broadcasted_iota(jnp.int32, (W, W), 0)
        d = (i_iota - j_iota).astype(jnp.float32)
        lane = jax.lax.broadcasted_iota(jnp.int32, (1, W), 1).astype(jnp.float32)
        for k in range(kk):
            sk = s_row[0:1, k:k + 1]
            la = la_row[0:1, k:k + 1]
            u_ref[:, k * W:(k + 1) * W] = jnp.where(i_iota >= j_iota,
                                                    sk * jnp.exp(d * la), 0.0)
            p_ref[k:k + 1, :] = jnp.exp((lane + 1.0) * la)
        m_ref[...] = jnp.broadcast_to(x2[:, 0:1], m_ref.shape)

    s_all = jnp.dot(x2, u_ref[...], preferred_element_type=jnp.float32)

    x3 = x_ref[...]
    alpha = jnp.exp(alpha_ref[...])[0][None, :, None]
    delta = jnp.exp(delta_ref[...])[0][None, :, None]
    r = jnp.exp(r_ref[...])[0][None, :, None]
    delta_r = jnp.exp(r * jnp.log(delta))

    for k in range(kk):
        m = s_all[:, k * W:(k + 1) * W] + m_ref[:, k:k + 1] * p_ref[k:k + 1, :]
        m_ref[:, k:k + 1] = m[:, W - 1:W]
        m3 = m.reshape(n_b, n_f, W)
        smooth = jnp.exp(-alpha * jnp.log(m3 + EPS))
        y = x3 * smooth + delta
        o_ref[:, k, :, :] = jnp.exp(r * jnp.log(y)) - delta_r


def kernel(x, s_log, alpha_log, delta_log, r_log):
    b, f, t = x.shape
    kk = s_log.shape[0]
    return pl.pallas_call(
        _pcen_kernel,
        grid=(b // B_BLK, t // W),
        in_specs=[
            pl.BlockSpec((1, kk), lambda i, j: (0, 0)),
            pl.BlockSpec((1, f), lambda i, j: (0, 0)),
            pl.BlockSpec((1, f), lambda i, j: (0, 0)),
            pl.BlockSpec((1, f), lambda i, j: (0, 0)),
            pl.BlockSpec((B_BLK, f, W), lambda i, j: (i, 0, j)),
        ],
        out_specs=pl.BlockSpec((B_BLK, kk, f, W), lambda i, j: (i, 0, 0, j)),
        out_shape=jax.ShapeDtypeStruct((b, kk, f, t), jnp.float32),
        scratch_shapes=[
            pltpu.VMEM((W, kk * W), jnp.float32),
            pltpu.VMEM((16, W), jnp.float32),
            pltpu.VMEM((B_BLK * f, 128), jnp.float32),
        ],
        compiler_params=pltpu.CompilerParams(
            dimension_semantics=("parallel", "arbitrary"),
        ),
        name="pcen10_fused",
    )(s_log.reshape(1, kk), alpha_log.reshape(1, f),
      delta_log.reshape(1, f), r_log.reshape(1, f), x)


# speedup vs baseline: 34.0764x; 1.2459x over previous
import jax
import jax.numpy as jnp
from jax.experimental import pallas as pl
from jax.experimental.pallas import tpu as pltpu

EPS = 1e-05
W = 128
B_BLK = 8


def _pcen_kernel(s_ref, alpha_ref, delta_ref, r_ref, x_ref, o_ref,
                 u_ref, xa_ref):
    t = pl.program_id(1)
    n_b, n_f, _ = x_ref.shape
    bf = n_b * n_f
    kk = s_ref.shape[1]
    nc = kk * W

    x2 = x_ref[...].reshape(bf, W)

    @pl.when(t == 0)
    def _init():
        s_row = jnp.exp(s_ref[...])
        la_row = jnp.log1p(-s_row)
        i_iota = jax.lax.broadcasted_iota(jnp.int32, (W, W), 1)
        j_iota = jax.lax.broadcasted_iota(jnp.int32, (W, W), 0)
        d = (i_iota - j_iota).astype(jnp.float32)
        lane = jax.lax.broadcasted_iota(jnp.int32, (1, W), 1)
        lane_f = lane.astype(jnp.float32)
        col_f = jax.lax.broadcasted_iota(jnp.int32, (W, 1), 0).astype(jnp.float32)
        u_ref[...] = jnp.zeros_like(u_ref)
        for k in range(kk):
            sk = s_row[0:1, k:k + 1]
            la = la_row[0:1, k:k + 1]
            u_ref[W:, k * W:(k + 1) * W] = jnp.where(i_iota >= j_iota,
                                                     sk * jnp.exp(d * la), 0.0)
            u_ref[k:k + 1, k * W:(k + 1) * W] = jnp.exp((lane_f + 1.0) * la)
            u_ref[W:, nc + k:nc + k + 1] = sk * jnp.exp((W - 1.0 - col_f) * la)
            u_ref[k:k + 1, nc:] = jnp.where(lane == k,
                                            jnp.exp(jnp.float32(W) * la), 0.0)
        xa_ref[:, 0:W] = jnp.broadcast_to(x2[:, 0:1], (bf, W))

    xa_ref[:, W:] = x2
    m_all = jnp.dot(xa_ref[...], u_ref[...], preferred_element_type=jnp.float32)
    xa_ref[:, 0:W] = m_all[:, nc:]

    x3 = x_ref[...]
    alpha = jnp.exp(alpha_ref[...])[0][None, :, None]
    delta = jnp.exp(delta_ref[...])[0][None, :, None]
    r = jnp.exp(r_ref[...])[0][None, :, None]
    delta_r = jnp.exp(r * jnp.log(delta))

    for k in range(kk):
        m3 = m_all[:, k * W:(k + 1) * W].reshape(n_b, n_f, W)
        smooth = jnp.exp(-alpha * jnp.log(m3 + EPS))
        y = x3 * smooth + delta
        o_ref[:, k, :, :] = jnp.exp(r * jnp.log(y)) - delta_r


def kernel(x, s_log, alpha_log, delta_log, r_log):
    b, f, t = x.shape
    kk = s_log.shape[0]
    return pl.pallas_call(
        _pcen_kernel,
        grid=(b // B_BLK, t // W),
        in_specs=[
            pl.BlockSpec((1, kk), lambda i, j: (0, 0)),
            pl.BlockSpec((1, f), lambda i, j: (0, 0)),
            pl.BlockSpec((1, f), lambda i, j: (0, 0)),
            pl.BlockSpec((1, f), lambda i, j: (0, 0)),
            pl.BlockSpec((B_BLK, f, W), lambda i, j: (i, 0, j)),
        ],
        out_specs=pl.BlockSpec((B_BLK, kk, f, W), lambda i, j: (i, 0, 0, j)),
        out_shape=jax.ShapeDtypeStruct((b, kk, f, t), jnp.float32),
        scratch_shapes=[
            pltpu.VMEM((2 * W, kk * W + W), jnp.float32),
            pltpu.VMEM((B_BLK * f, 2 * W), jnp.float32),
        ],
        compiler_params=pltpu.CompilerParams(
            dimension_semantics=("parallel", "arbitrary"),
        ),
        name="pcen10_fused",
    )(s_log.reshape(1, kk), alpha_log.reshape(1, f),
      delta_log.reshape(1, f), r_log.reshape(1, f), x)


# speedup vs baseline: 34.1370x; 1.0018x over previous
import jax
import jax.numpy as jnp
from jax.experimental import pallas as pl
from jax.experimental.pallas import tpu as pltpu

EPS = 1e-05
W = 128
B_BLK = 8


def _pcen_kernel(s_ref, alpha_ref, delta_ref, r_ref, x_ref, o_ref,
                 u_ref, xa_ref):
    t = pl.program_id(1)
    n_b, n_f, _ = x_ref.shape
    bf = n_b * n_f
    kk = s_ref.shape[1]
    nc = kk * W

    x2 = x_ref[...].reshape(bf, W)

    @pl.when(t == 0)
    def _init():
        s_row = jnp.exp(s_ref[...])
        la_row = jnp.log1p(-s_row)
        i_iota = jax.lax.broadcasted_iota(jnp.int32, (W, W), 1)
        j_iota = jax.lax.broadcasted_iota(jnp.int32, (W, W), 0)
        d = (i_iota - j_iota).astype(jnp.float32)
        lane = jax.lax.broadcasted_iota(jnp.int32, (1, W), 1)
        lane_f = lane.astype(jnp.float32)
        col_f = jax.lax.broadcasted_iota(jnp.int32, (W, 1), 0).astype(jnp.float32)
        u_ref[...] = jnp.zeros_like(u_ref)
        for k in range(kk):
            sk = s_row[0:1, k:k + 1]
            la = la_row[0:1, k:k + 1]
            u_ref[W:, k * W:(k + 1) * W] = jnp.where(i_iota >= j_iota,
                                                     sk * jnp.exp(d * la), 0.0)
            u_ref[k:k + 1, k * W:(k + 1) * W] = jnp.exp((lane_f + 1.0) * la)
            u_ref[W:, nc + k:nc + k + 1] = sk * jnp.exp((W - 1.0 - col_f) * la)
            u_ref[k:k + 1, nc:] = jnp.where(lane == k,
                                            jnp.exp(jnp.float32(W) * la), 0.0)
        xa_ref[:, 0:W] = jnp.broadcast_to(x2[:, 0:1], (bf, W))

    xa_ref[:, W:] = x2
    m_all = jnp.dot(xa_ref[...], u_ref[...], preferred_element_type=jnp.float32)
    xa_ref[:, 0:W] = m_all[:, nc:]

    x3 = x_ref[...]
    alpha = jnp.exp(alpha_ref[...])[0][None, :, None]
    delta = jnp.exp(delta_ref[...])[0][None, :, None]
    r = jnp.exp(r_ref[...])[0][None, :, None]
    delta_r = jnp.exp2(r * jnp.log2(delta))

    for k in range(kk):
        m3 = m_all[:, k * W:(k + 1) * W].reshape(n_b, n_f, W)
        smooth = jnp.exp2(-alpha * jnp.log2(m3 + EPS))
        y = x3 * smooth + delta
        o_ref[:, k, :, :] = jnp.exp2(r * jnp.log2(y)) - delta_r


def kernel(x, s_log, alpha_log, delta_log, r_log):
    b, f, t = x.shape
    kk = s_log.shape[0]
    return pl.pallas_call(
        _pcen_kernel,
        grid=(b // B_BLK, t // W),
        in_specs=[
            pl.BlockSpec((1, kk), lambda i, j: (0, 0)),
            pl.BlockSpec((1, f), lambda i, j: (0, 0)),
            pl.BlockSpec((1, f), lambda i, j: (0, 0)),
            pl.BlockSpec((1, f), lambda i, j: (0, 0)),
            pl.BlockSpec((B_BLK, f, W), lambda i, j: (i, 0, j)),
        ],
        out_specs=pl.BlockSpec((B_BLK, kk, f, W), lambda i, j: (i, 0, 0, j)),
        out_shape=jax.ShapeDtypeStruct((b, kk, f, t), jnp.float32),
        scratch_shapes=[
            pltpu.VMEM((2 * W, kk * W + W), jnp.float32),
            pltpu.VMEM((B_BLK * f, 2 * W), jnp.float32),
        ],
        compiler_params=pltpu.CompilerParams(
            dimension_semantics=("parallel", "arbitrary"),
        ),
        name="pcen10_fused",
    )(s_log.reshape(1, kk), alpha_log.reshape(1, f),
      delta_log.reshape(1, f), r_log.reshape(1, f), x)
